# Initial kernel scaffold; baseline (speedup 1.0000x reference)
import jax
import jax.numpy as jnp
from jax.experimental import pallas as pl


def kernel(x_embeddings, edge_index, weights, batch, W1_rel, b1_rel, W1_root, W2_rel, b2_rel, W2_root, W_lin_root):
    raise NotImplementedError("write your pallas kernel here")


# speedup vs baseline: 9.8703x; 9.8703x over previous
import jax
import jax.numpy as jnp
from jax import lax
from jax.experimental import pallas as pl
from jax.experimental.pallas import tpu as pltpu
from jax.experimental.pallas import tpu_sc as plsc

N = 10000
E = 320000
D = 128
H = 128
C = 8
B = 64

NC = 2
NS = 16
NW = NC * NS
KB = 128
BPW = 80
EPW = KB * BPW
E_PAD = NW * EPW
N_PAD = 10240
RPS = N_PAD // NS
BLK = 1024
GRID = N_PAD // BLK


def _sc_agg_body(x_hbm, sdw_hbm, zero_hbm, out_hbm,
                 idx0, idx1, idx2, idx3, rows0, rows1, agg_sh,
                 isem0, isem1, isem2, isem3, gsem0, gsem1, ssem0, ssem1):
    c = lax.axis_index("c")
    s = lax.axis_index("s")
    w = c * NS + s

    pltpu.sync_copy(zero_hbm.at[pl.ds(s * RPS, RPS)],
                    agg_sh.at[pl.ds(s * RPS, RPS)])

    idx = (idx0, idx1, idx2, idx3)
    isem = (isem0, isem1, isem2, isem3)
    rows = (rows0, rows1)
    gsem = (gsem0, gsem1)
    ssem = (ssem0, ssem1)
    two = jnp.full((16,), 2, jnp.int32)

    def i_start(b, r):
        pltpu.async_copy(sdw_hbm.at[w, b], idx[r], isem[r])

    def i_wait(b, r):
        pltpu.make_async_copy(sdw_hbm.at[w, b], idx[r], isem[r]).wait()

    def g_start(b, r, p):
        pltpu.async_copy(x_hbm.at[idx[r].at[0]], rows[p], gsem[p])

    def g_wait(b, r, p):
        pltpu.make_async_copy(x_hbm.at[idx[r].at[0]], rows[p], gsem[p]).wait()

    def s_start(b, r, p):
        pltpu.async_copy(rows[p], agg_sh.at[idx[r].at[1]], ssem[p], add=True)

    def s_wait(b, r, p):
        pltpu.make_async_copy(rows[p], agg_sh.at[idx[r].at[1]], ssem[p]).wait()

    def scale(r, p):
        buf = rows[p]
        iref = idx[r]

        @plsc.parallel_loop(0, KB)
        def _(e):
            wi = plsc.load_gather(iref, [two, jnp.full((16,), e, jnp.int32)])
            wv = plsc.bitcast(wi, jnp.float32)
            for j in range(8):
                sl = pl.ds(j * 16, 16)
                buf[e, sl] = buf[e, sl] * wv

    def body(b, r, p, first=False, lastb=False, last3=False):
        g_wait(b, r, p)
        if not first:
            s_wait(b - 1, (r - 1) % 4, 1 - p)
        if not lastb:
            i_wait(b + 1, (r + 1) % 4)
            g_start(b + 1, (r + 1) % 4, 1 - p)
        if not last3:
            i_start(b + 3, (r + 3) % 4)
        scale(r, p)
        s_start(b, r, p)

    i_start(0, 0)
    i_start(1, 1)
    i_start(2, 2)
    i_wait(0, 0)
    plsc.subcore_barrier()
    g_start(0, 0, 0)
    body(0, 0, 0, first=True)

    @pl.loop(1, BPW - 6, step=4)
    def _(t):
        for j in range(4):
            body(t + j, (1 + j) % 4, (1 + j) % 2)

    body(BPW - 3, (BPW - 3) % 4, (BPW - 3) % 2, last3=True)
    body(BPW - 2, (BPW - 2) % 4, (BPW - 2) % 2, last3=True)
    body(BPW - 1, (BPW - 1) % 4, (BPW - 1) % 2, last3=True, lastb=True)
    s_wait(BPW - 1, (BPW - 1) % 4, (BPW - 1) % 2)

    plsc.subcore_barrier()
    pltpu.sync_copy(agg_sh.at[pl.ds(s * RPS, RPS)],
                    out_hbm.at[c, pl.ds(s * RPS, RPS)])


import functools


@functools.cache
def _make_sc_agg():
  return pl.kernel(
    _sc_agg_body,
    out_type=jax.ShapeDtypeStruct((NC, N_PAD, D), jnp.float32),
    mesh=plsc.VectorSubcoreMesh(core_axis_name="c", subcore_axis_name="s",
                                num_cores=NC, num_subcores=NS),
    scratch_types=[
        pltpu.VMEM((3, KB), jnp.int32),
        pltpu.VMEM((3, KB), jnp.int32),
        pltpu.VMEM((3, KB), jnp.int32),
        pltpu.VMEM((3, KB), jnp.int32),
        pltpu.VMEM((KB, D), jnp.float32),
        pltpu.VMEM((KB, D), jnp.float32),
        pltpu.VMEM_SHARED((N_PAD, D), jnp.float32),
        pltpu.SemaphoreType.DMA,
        pltpu.SemaphoreType.DMA,
        pltpu.SemaphoreType.DMA,
        pltpu.SemaphoreType.DMA,
        pltpu.SemaphoreType.DMA,
        pltpu.SemaphoreType.DMA,
        pltpu.SemaphoreType.DMA,
        pltpu.SemaphoreType.DMA,
    ],
    compiler_params=pltpu.CompilerParams(needs_layout_passes=False),
  )


def _layer_body(a0, a1, x, wr, wroot, bias, o):
    agg = a0[...] + a1[...]
    acc = jnp.dot(agg, wr[...], preferred_element_type=jnp.float32)
    acc = acc + jnp.dot(x[...], wroot[...], preferred_element_type=jnp.float32)
    acc = acc + bias[...]
    o[...] = jnp.where(acc >= 0.0, acc, 0.01 * acc)


_layer = pl.pallas_call(
    _layer_body,
    grid=(GRID,),
    in_specs=[
        pl.BlockSpec((BLK, D), lambda i: (i, 0)),
        pl.BlockSpec((BLK, D), lambda i: (i, 0)),
        pl.BlockSpec((BLK, D), lambda i: (i, 0)),
        pl.BlockSpec((D, H), lambda i: (0, 0)),
        pl.BlockSpec((D, H), lambda i: (0, 0)),
        pl.BlockSpec((1, H), lambda i: (0, 0)),
    ],
    out_specs=pl.BlockSpec((BLK, H), lambda i: (i, 0)),
    out_shape=jax.ShapeDtypeStruct((N_PAD, H), jnp.float32),
)


def _final_body(a0, a1, x1, wr, wroot, bias, bt, wlin, o, sums, counts):
    i = pl.program_id(0)

    @pl.when(i == 0)
    def _():
        sums[...] = jnp.zeros_like(sums)
        counts[...] = jnp.zeros_like(counts)

    agg = a0[...] + a1[...]
    acc = jnp.dot(agg, wr[...], preferred_element_type=jnp.float32)
    acc = acc + jnp.dot(x1[...], wroot[...], preferred_element_type=jnp.float32)
    acc = acc + bias[...]
    x2 = jnp.where(acc >= 0.0, acc, 0.01 * acc)

    bids = bt[0, 0, :]
    m = (bids[None, :] == lax.broadcasted_iota(jnp.int32, (B, BLK), 0)
         ).astype(jnp.float32)
    sums[...] = sums[...] + jnp.dot(m, x2, preferred_element_type=jnp.float32)
    counts[...] = counts[...] + jnp.sum(m, axis=1, keepdims=True)

    @pl.when(i == GRID - 1)
    def _():
        pooled = sums[...] / jnp.maximum(counts[...], 1.0)
        o[...] = jnp.dot(pooled, wlin[...], preferred_element_type=jnp.float32)


_final = pl.pallas_call(
    _final_body,
    grid=(GRID,),
    in_specs=[
        pl.BlockSpec((BLK, H), lambda i: (i, 0)),
        pl.BlockSpec((BLK, H), lambda i: (i, 0)),
        pl.BlockSpec((BLK, H), lambda i: (i, 0)),
        pl.BlockSpec((H, H), lambda i: (0, 0)),
        pl.BlockSpec((H, H), lambda i: (0, 0)),
        pl.BlockSpec((1, H), lambda i: (0, 0)),
        pl.BlockSpec((1, 1, BLK), lambda i: (i, 0, 0)),
        pl.BlockSpec((H, 128), lambda i: (0, 0)),
    ],
    out_specs=pl.BlockSpec((B, 128), lambda i: (0, 0)),
    out_shape=jax.ShapeDtypeStruct((B, 128), jnp.float32),
    scratch_shapes=[
        pltpu.VMEM((B, H), jnp.float32),
        pltpu.VMEM((B, 128), jnp.float32),
    ],
)


def kernel(x_embeddings, edge_index, weights, batch,
           W1_rel, b1_rel, W1_root, W2_rel, b2_rel, W2_root, W_lin_root):
    src = edge_index[0]
    dst = edge_index[1]
    pad_e = E_PAD - E
    fill = jnp.arange(pad_e, dtype=jnp.int32) % N
    src_p = jnp.concatenate([src, fill]).reshape(NW, BPW, 1, KB)
    dst_p = jnp.concatenate([dst, fill]).reshape(NW, BPW, 1, KB)
    ew_i = lax.bitcast_convert_type(
        jnp.concatenate([weights, jnp.zeros((pad_e,), jnp.float32)]),
        jnp.int32).reshape(NW, BPW, 1, KB)
    sdw = jnp.concatenate([src_p, dst_p, ew_i], axis=2)
    x_p = jnp.pad(x_embeddings, ((0, N_PAD - N), (0, 0)))
    zeros = jnp.zeros((N_PAD, D), jnp.float32)
    batch_p = jnp.pad(batch, (0, N_PAD - N),
                      constant_values=B).reshape(GRID, 1, BLK)
    wlin_p = jnp.pad(W_lin_root, ((0, 0), (0, 128 - C)))

    sc_agg = _make_sc_agg()
    agg1 = sc_agg(x_p, sdw, zeros)
    x1 = _layer(agg1[0], agg1[1], x_p, W1_rel, W1_root, b1_rel.reshape(1, H))
    agg2 = sc_agg(x1, sdw, zeros)
    out = _final(agg2[0], agg2[1], x1, W2_rel, W2_root,
                 b2_rel.reshape(1, H), batch_p, wlin_p)
    return out[:, :C]


# speedup vs baseline: 10.3401x; 1.0476x over previous
import functools

import jax
import jax.numpy as jnp
from jax import lax
from jax.experimental import pallas as pl
from jax.experimental.pallas import tpu as pltpu
from jax.experimental.pallas import tpu_sc as plsc

N = 10000
E = 320000
D = 128
H = 128
C = 8
B = 64

NC = 2
NS = 16
NW = NC * NS
KB = 128
BPW = 80
EPW = KB * BPW
E_PAD = NW * EPW
N_PAD = 10240
RPS = N_PAD // NS
BLK = 1024
GRID = N_PAD // BLK


def _sc_agg_body(x_hbm, sdw_hbm, zero_hbm, out_hbm,
                 idx0, idx1, idx2, idx3, rows0, rows1, agg_sh,
                 isem0, isem1, isem2, isem3, gsem0, gsem1, ssem0, ssem1):
    c = lax.axis_index("c")
    s = lax.axis_index("s")
    w = c * NS + s

    idx = (idx0, idx1, idx2, idx3)
    isem = (isem0, isem1, isem2, isem3)
    rows = (rows0, rows1)
    gsem = (gsem0, gsem1)
    ssem = (ssem0, ssem1)
    two = jnp.full((16,), 2, jnp.int32)

    def i_start(b, r):
        pltpu.async_copy(sdw_hbm.at[w, b], idx[r], isem[r])

    def i_wait(b, r):
        pltpu.make_async_copy(sdw_hbm.at[w, b], idx[r], isem[r]).wait()

    def g_start(b, r, p):
        pltpu.async_copy(x_hbm.at[idx[r].at[0]], rows[p], gsem[p])

    def g_wait(b, r, p):
        pltpu.make_async_copy(x_hbm.at[idx[r].at[0]], rows[p], gsem[p]).wait()

    def s_start(b, r, p):
        pltpu.async_copy(rows[p], agg_sh.at[idx[r].at[1]], ssem[p], add=True)

    def s_wait(b, r, p):
        pltpu.make_async_copy(rows[p], agg_sh.at[idx[r].at[1]], ssem[p]).wait()

    def scale(r, p):
        buf = rows[p]
        iref = idx[r]

        @plsc.parallel_loop(0, KB, unroll=2)
        def _(e):
            wi = plsc.load_gather(iref, [two, jnp.full((16,), e, jnp.int32)])
            wv = plsc.bitcast(wi, jnp.float32)
            for j in range(8):
                sl = pl.ds(j * 16, 16)
                buf[e, sl] = buf[e, sl] * wv

    def body(b, r, p, first=False, lastb=False, last3=False):
        g_wait(b, r, p)
        if not first:
            s_wait(b - 1, (r - 1) % 4, 1 - p)
        if not lastb:
            i_wait(b + 1, (r + 1) % 4)
            g_start(b + 1, (r + 1) % 4, 1 - p)
        if not last3:
            i_start(b + 3, (r + 3) % 4)
        scale(r, p)
        s_start(b, r, p)

    i_start(0, 0)
    i_start(1, 1)
    i_start(2, 2)
    pltpu.sync_copy(zero_hbm.at[pl.ds(s * RPS, RPS)],
                    agg_sh.at[pl.ds(s * RPS, RPS)])
    i_wait(0, 0)
    g_start(0, 0, 0)
    plsc.subcore_barrier()
    body(0, 0, 0, first=True)

    @pl.loop(1, BPW - 6, step=4)
    def _(t):
        for j in range(4):
            body(t + j, (1 + j) % 4, (1 + j) % 2)

    body(BPW - 3, (BPW - 3) % 4, (BPW - 3) % 2, last3=True)
    body(BPW - 2, (BPW - 2) % 4, (BPW - 2) % 2, last3=True)
    body(BPW - 1, (BPW - 1) % 4, (BPW - 1) % 2, last3=True, lastb=True)
    s_wait(BPW - 1, (BPW - 1) % 4, (BPW - 1) % 2)

    plsc.subcore_barrier()
    pltpu.sync_copy(agg_sh.at[pl.ds(s * RPS, RPS)],
                    out_hbm.at[c, pl.ds(s * RPS, RPS)])


@functools.cache
def _make_sc_agg():
  return pl.kernel(
    _sc_agg_body,
    out_type=jax.ShapeDtypeStruct((NC, N_PAD, D), jnp.float32),
    mesh=plsc.VectorSubcoreMesh(core_axis_name="c", subcore_axis_name="s",
                                num_cores=NC, num_subcores=NS),
    scratch_types=[
        pltpu.VMEM((3, KB), jnp.int32),
        pltpu.VMEM((3, KB), jnp.int32),
        pltpu.VMEM((3, KB), jnp.int32),
        pltpu.VMEM((3, KB), jnp.int32),
        pltpu.VMEM((KB, D), jnp.float32),
        pltpu.VMEM((KB, D), jnp.float32),
        pltpu.VMEM_SHARED((N_PAD, D), jnp.float32),
        pltpu.SemaphoreType.DMA,
        pltpu.SemaphoreType.DMA,
        pltpu.SemaphoreType.DMA,
        pltpu.SemaphoreType.DMA,
        pltpu.SemaphoreType.DMA,
        pltpu.SemaphoreType.DMA,
        pltpu.SemaphoreType.DMA,
        pltpu.SemaphoreType.DMA,
    ],
    compiler_params=pltpu.CompilerParams(needs_layout_passes=False),
  )


def _layer_body(a, x, wr, wroot, bias, o):
    agg = a[0] + a[1]
    acc = jnp.dot(agg, wr[...], preferred_element_type=jnp.float32)
    acc = acc + jnp.dot(x[...], wroot[...], preferred_element_type=jnp.float32)
    acc = acc + bias[...]
    o[...] = jnp.where(acc >= 0.0, acc, 0.01 * acc)


_layer = pl.pallas_call(
    _layer_body,
    grid=(GRID,),
    in_specs=[
        pl.BlockSpec((2, BLK, D), lambda i: (0, i, 0)),
        pl.BlockSpec((BLK, D), lambda i: (i, 0)),
        pl.BlockSpec((D, H), lambda i: (0, 0)),
        pl.BlockSpec((D, H), lambda i: (0, 0)),
        pl.BlockSpec((1, H), lambda i: (0, 0)),
    ],
    out_specs=pl.BlockSpec((BLK, H), lambda i: (i, 0)),
    out_shape=jax.ShapeDtypeStruct((N_PAD, H), jnp.float32),
)


def _final_body(a, x1, wr, wroot, bias, bt, wlin, o, sums, counts):
    i = pl.program_id(0)

    @pl.when(i == 0)
    def _():
        sums[...] = jnp.zeros_like(sums)
        counts[...] = jnp.zeros_like(counts)

    agg = a[0] + a[1]
    acc = jnp.dot(agg, wr[...], preferred_element_type=jnp.float32)
    acc = acc + jnp.dot(x1[...], wroot[...], preferred_element_type=jnp.float32)
    acc = acc + bias[...]
    x2 = jnp.where(acc >= 0.0, acc, 0.01 * acc)

    bids = bt[0, 0, :]
    m = (bids[None, :] == lax.broadcasted_iota(jnp.int32, (B, BLK), 0)
         ).astype(jnp.float32)
    sums[...] = sums[...] + jnp.dot(m, x2, preferred_element_type=jnp.float32)
    counts[...] = counts[...] + jnp.sum(m, axis=1, keepdims=True)

    @pl.when(i == GRID - 1)
    def _():
        pooled = sums[...] / jnp.maximum(counts[...], 1.0)
        o[...] = jnp.dot(pooled, wlin[...], preferred_element_type=jnp.float32)


_final = pl.pallas_call(
    _final_body,
    grid=(GRID,),
    in_specs=[
        pl.BlockSpec((2, BLK, H), lambda i: (0, i, 0)),
        pl.BlockSpec((BLK, H), lambda i: (i, 0)),
        pl.BlockSpec((H, H), lambda i: (0, 0)),
        pl.BlockSpec((H, H), lambda i: (0, 0)),
        pl.BlockSpec((1, H), lambda i: (0, 0)),
        pl.BlockSpec((1, 1, BLK), lambda i: (i, 0, 0)),
        pl.BlockSpec((H, 128), lambda i: (0, 0)),
    ],
    out_specs=pl.BlockSpec((B, 128), lambda i: (0, 0)),
    out_shape=jax.ShapeDtypeStruct((B, 128), jnp.float32),
    scratch_shapes=[
        pltpu.VMEM((B, H), jnp.float32),
        pltpu.VMEM((B, 128), jnp.float32),
    ],
)


def kernel(x_embeddings, edge_index, weights, batch,
           W1_rel, b1_rel, W1_root, W2_rel, b2_rel, W2_root, W_lin_root):
    src = edge_index[0]
    dst = edge_index[1]
    pad_e = E_PAD - E
    fill = jnp.arange(pad_e, dtype=jnp.int32) % N
    src_p = jnp.concatenate([src, fill]).reshape(NW, BPW, 1, KB)
    dst_p = jnp.concatenate([dst, fill]).reshape(NW, BPW, 1, KB)
    ew_i = lax.bitcast_convert_type(
        jnp.concatenate([weights, jnp.zeros((pad_e,), jnp.float32)]),
        jnp.int32).reshape(NW, BPW, 1, KB)
    sdw = jnp.concatenate([src_p, dst_p, ew_i], axis=2)
    x_p = jnp.pad(x_embeddings, ((0, N_PAD - N), (0, 0)))

    zeros = jnp.zeros((N_PAD, D), jnp.float32)
    batch_p = jnp.pad(batch, (0, N_PAD - N),
                      constant_values=B).reshape(GRID, 1, BLK)
    wlin_p = jnp.pad(W_lin_root, ((0, 0), (0, 128 - C)))

    sc_agg = _make_sc_agg()
    agg1 = sc_agg(x_p, sdw, zeros)
    x1 = _layer(agg1, x_p, W1_rel, W1_root, b1_rel.reshape(1, H))
    agg2 = sc_agg(x1, sdw, zeros)
    out = _final(agg2, x1, W2_rel, W2_root,
                 b2_rel.reshape(1, H), batch_p, wlin_p)
    return out[:, :C]
